# baseline (speedup 1.0000x reference)
import functools

import jax
import jax.numpy as jnp
from jax import lax
from jax.experimental import pallas as pl
from jax.experimental.pallas import tpu as pltpu
from jax.experimental.pallas import tpu_sc as plsc

_VOCAB = 100000
_EMBED_DIM = 16
_BATCH = 1024
_VT = 2048


@functools.cache
def _make_sc_gather():
    info = plsc.get_sparse_core_info()
    nw = info.num_cores * info.num_subcores
    b_per_w = _BATCH // nw
    mesh = plsc.VectorSubcoreMesh(core_axis_name="c", subcore_axis_name="s")

    @functools.partial(
        pl.kernel,
        mesh=mesh,
        out_type=jax.ShapeDtypeStruct((_EMBED_DIM + 1, _BATCH), jnp.float32),
        scratch_types=[
            pltpu.VMEM((_EMBED_DIM, b_per_w), jnp.int32),
            pltpu.VMEM((_EMBED_DIM + 1, b_per_w), jnp.float32),
            pltpu.SemaphoreType.DMA,
        ],
        compiler_params=pltpu.CompilerParams(use_tc_tiling_on_sc=False),
    )
    def gather_kernel(table_hbm, offs_hbm, out_hbm, offs_v, rows_v, sem):
        wid = lax.axis_index("s") * info.num_cores + lax.axis_index("c")
        base = wid * b_per_w
        pltpu.sync_copy(offs_hbm.at[:, pl.ds(base, b_per_w)], offs_v)
        copies = [
            pltpu.async_copy(table_hbm.at[offs_v.at[d]], rows_v.at[d], sem)
            for d in range(_EMBED_DIM)
        ]
        for h in range(b_per_w // 16):
            rows_v[_EMBED_DIM, pl.ds(h * 16, 16)] = jnp.full(
                (16,), 1.0, jnp.float32)
        for c in copies:
            c.wait()
        pltpu.sync_copy(rows_v, out_hbm.at[:, pl.ds(base, b_per_w)])

    return gather_kernel


_KA = _EMBED_DIM + 1


def _tn_dot(wta, xta, precision=None):
    return jax.lax.dot_general(
        wta, xta, (((0,), (0,)), ((), ())),
        precision=precision,
        preferred_element_type=jnp.float32)


def _lse_body(wta_ref, xta_ref, lse_ref, s_acc):
    j = pl.program_id(0)
    lt = _tn_dot(wta_ref[...], xta_ref[...],
                 precision=jax.lax.Precision.DEFAULT)
    vids = jax.lax.broadcasted_iota(jnp.int32, (_VT, 1), 0) + j * _VT
    lt = jnp.where(vids < _VOCAB, lt, jnp.float32(-1e30))
    ssum = jnp.sum(jnp.exp(lt), axis=0, keepdims=True)

    @pl.when(j == 0)
    def _init():
        s_acc[...] = ssum

    @pl.when(j > 0)
    def _accum():
        s_acc[...] += ssum

    @pl.when(j == pl.num_programs(0) - 1)
    def _final():
        lse_ref[...] = jnp.log(s_acc[...])


def _out_body(wta_ref, xta_ref, lse_ref, out_ref):
    lt = _tn_dot(wta_ref[...], xta_ref[...])
    out_ref[...] = lt - lse_ref[...]


def kernel(inputs, emb_table, W, b):
    idx = inputs.astype(jnp.int32)
    table_lin = emb_table.T.reshape(-1)
    offs = (jnp.arange(_EMBED_DIM, dtype=jnp.int32) * _VOCAB)[:, None] + idx[None, :]
    xta = _make_sc_gather()(table_lin, offs)
    wta = jnp.concatenate([W.T, b.reshape(1, _VOCAB)], axis=0)
    grid = (pl.cdiv(_VOCAB, _VT),)
    w_spec = pl.BlockSpec((_KA, _VT), lambda j: (0, j))
    xt_spec = pl.BlockSpec((_KA, _BATCH), lambda j: (0, 0))
    lse_spec = pl.BlockSpec((1, _BATCH), lambda j: (0, 0))
    lse = pl.pallas_call(
        _lse_body,
        grid=grid,
        in_specs=[w_spec, xt_spec],
        out_specs=lse_spec,
        out_shape=jax.ShapeDtypeStruct((1, _BATCH), jnp.float32),
        scratch_shapes=[pltpu.VMEM((1, _BATCH), jnp.float32)],
    )(wta, xta)
    out_t = pl.pallas_call(
        _out_body,
        grid=grid,
        in_specs=[w_spec, xt_spec, lse_spec],
        out_specs=pl.BlockSpec((_VT, _BATCH), lambda j: (j, 0)),
        out_shape=jax.ShapeDtypeStruct((_VOCAB, _BATCH), jnp.float32),
    )(wta, xta, lse)
    return out_t.T


# speedup vs baseline: 1.0205x; 1.0205x over previous
import functools

import jax
import jax.numpy as jnp
from jax import lax
from jax.experimental import pallas as pl
from jax.experimental.pallas import tpu as pltpu
from jax.experimental.pallas import tpu_sc as plsc

_VOCAB = 100000
_EMBED_DIM = 16
_BATCH = 1024
_VT = 2048
_VTA = 4096


@functools.cache
def _make_sc_gather():
    info = plsc.get_sparse_core_info()
    nw = info.num_cores * info.num_subcores
    b_per_w = _BATCH // nw
    mesh = plsc.VectorSubcoreMesh(core_axis_name="c", subcore_axis_name="s")

    @functools.partial(
        pl.kernel,
        mesh=mesh,
        out_type=jax.ShapeDtypeStruct((_EMBED_DIM + 1, _BATCH), jnp.float32),
        scratch_types=[
            pltpu.VMEM((_EMBED_DIM, b_per_w), jnp.int32),
            pltpu.VMEM((_EMBED_DIM + 1, b_per_w), jnp.float32),
            pltpu.SemaphoreType.DMA,
        ],
        compiler_params=pltpu.CompilerParams(use_tc_tiling_on_sc=False),
    )
    def gather_kernel(table_hbm, offs_hbm, out_hbm, offs_v, rows_v, sem):
        wid = lax.axis_index("s") * info.num_cores + lax.axis_index("c")
        base = wid * b_per_w
        pltpu.sync_copy(offs_hbm.at[:, pl.ds(base, b_per_w)], offs_v)
        copies = [
            pltpu.async_copy(table_hbm.at[offs_v.at[d]], rows_v.at[d], sem)
            for d in range(_EMBED_DIM)
        ]
        for h in range(b_per_w // 16):
            rows_v[_EMBED_DIM, pl.ds(h * 16, 16)] = jnp.full(
                (16,), 1.0, jnp.float32)
        for c in copies:
            c.wait()
        pltpu.sync_copy(rows_v, out_hbm.at[:, pl.ds(base, b_per_w)])

    return gather_kernel


_KA = _EMBED_DIM + 1


def _tn_dot(wta, xta, precision=None):
    return jax.lax.dot_general(
        wta, xta, (((0,), (0,)), ((), ())),
        precision=precision,
        preferred_element_type=jnp.float32)


def _lse_body(wta_ref, xta_ref, lse_ref, s_acc):
    j = pl.program_id(0)
    lt = _tn_dot(wta_ref[...], xta_ref[...],
                 precision=jax.lax.Precision.DEFAULT)
    vids = jax.lax.broadcasted_iota(jnp.int32, (_VTA, 1), 0) + j * _VTA
    lt = jnp.where(vids < _VOCAB, lt, jnp.float32(-1e30))
    ssum = jnp.sum(jnp.exp(lt), axis=0, keepdims=True)

    @pl.when(j == 0)
    def _init():
        s_acc[...] = ssum

    @pl.when(j > 0)
    def _accum():
        s_acc[...] += ssum

    @pl.when(j == pl.num_programs(0) - 1)
    def _final():
        lse_ref[...] = jnp.log(s_acc[...])


def _out_body(wta_ref, xta_ref, lse_ref, out_ref):
    lt = _tn_dot(wta_ref[...], xta_ref[...])
    out_ref[...] = lt - lse_ref[...]


def kernel(inputs, emb_table, W, b):
    idx = inputs.astype(jnp.int32)
    table_lin = emb_table.T.reshape(-1)
    offs = (jnp.arange(_EMBED_DIM, dtype=jnp.int32) * _VOCAB)[:, None] + idx[None, :]
    xta = _make_sc_gather()(table_lin, offs)
    wta = jnp.concatenate([W.T, b.reshape(1, _VOCAB)], axis=0)
    xt_spec = pl.BlockSpec((_KA, _BATCH), lambda j: (0, 0))
    lse_spec = pl.BlockSpec((1, _BATCH), lambda j: (0, 0))
    lse = pl.pallas_call(
        _lse_body,
        grid=(pl.cdiv(_VOCAB, _VTA),),
        in_specs=[pl.BlockSpec((_KA, _VTA), lambda j: (0, j)), xt_spec],
        out_specs=lse_spec,
        out_shape=jax.ShapeDtypeStruct((1, _BATCH), jnp.float32),
        scratch_shapes=[pltpu.VMEM((1, _BATCH), jnp.float32)],
    )(wta, xta)
    out_t = pl.pallas_call(
        _out_body,
        grid=(pl.cdiv(_VOCAB, _VT),),
        in_specs=[pl.BlockSpec((_KA, _VT), lambda j: (0, j)), xt_spec,
                  lse_spec],
        out_specs=pl.BlockSpec((_VT, _BATCH), lambda j: (j, 0)),
        out_shape=jax.ShapeDtypeStruct((_VOCAB, _BATCH), jnp.float32),
    )(wta, xta, lse)
    return out_t.T


# speedup vs baseline: 1.0216x; 1.0010x over previous
import functools

import jax
import jax.numpy as jnp
from jax import lax
from jax.experimental import pallas as pl
from jax.experimental.pallas import tpu as pltpu
from jax.experimental.pallas import tpu_sc as plsc

_VOCAB = 100000
_EMBED_DIM = 16
_BATCH = 1024
_VT = 2048
_VTA = 8192


@functools.cache
def _make_sc_gather():
    info = plsc.get_sparse_core_info()
    nw = info.num_cores * info.num_subcores
    b_per_w = _BATCH // nw
    mesh = plsc.VectorSubcoreMesh(core_axis_name="c", subcore_axis_name="s")

    @functools.partial(
        pl.kernel,
        mesh=mesh,
        out_type=jax.ShapeDtypeStruct((_EMBED_DIM + 1, _BATCH), jnp.float32),
        scratch_types=[
            pltpu.VMEM((_EMBED_DIM, b_per_w), jnp.int32),
            pltpu.VMEM((_EMBED_DIM + 1, b_per_w), jnp.float32),
            pltpu.SemaphoreType.DMA,
        ],
        compiler_params=pltpu.CompilerParams(use_tc_tiling_on_sc=False),
    )
    def gather_kernel(table_hbm, offs_hbm, out_hbm, offs_v, rows_v, sem):
        wid = lax.axis_index("s") * info.num_cores + lax.axis_index("c")
        base = wid * b_per_w
        pltpu.sync_copy(offs_hbm.at[:, pl.ds(base, b_per_w)], offs_v)
        copies = [
            pltpu.async_copy(table_hbm.at[offs_v.at[d]], rows_v.at[d], sem)
            for d in range(_EMBED_DIM)
        ]
        for h in range(b_per_w // 16):
            rows_v[_EMBED_DIM, pl.ds(h * 16, 16)] = jnp.full(
                (16,), 1.0, jnp.float32)
        for c in copies:
            c.wait()
        pltpu.sync_copy(rows_v, out_hbm.at[:, pl.ds(base, b_per_w)])

    return gather_kernel


_KA = _EMBED_DIM + 1


def _tn_dot(wta, xta, precision=None):
    return jax.lax.dot_general(
        wta, xta, (((0,), (0,)), ((), ())),
        precision=precision,
        preferred_element_type=jnp.float32)


def _lse_body(wta_ref, xta_ref, lse_ref, s_acc):
    j = pl.program_id(0)
    lt = _tn_dot(wta_ref[...], xta_ref[...],
                 precision=jax.lax.Precision.DEFAULT)
    vids = jax.lax.broadcasted_iota(jnp.int32, (_VTA, 1), 0) + j * _VTA
    lt = jnp.where(vids < _VOCAB, lt, jnp.float32(-1e30))
    ssum = jnp.sum(jnp.exp(lt), axis=0, keepdims=True)

    @pl.when(j == 0)
    def _init():
        s_acc[...] = ssum

    @pl.when(j > 0)
    def _accum():
        s_acc[...] += ssum

    @pl.when(j == pl.num_programs(0) - 1)
    def _final():
        lse_ref[...] = jnp.log(s_acc[...])


def _out_body(wta_ref, xta_ref, lse_ref, out_ref):
    lt = _tn_dot(wta_ref[...], xta_ref[...])
    out_ref[...] = lt - lse_ref[...]


def kernel(inputs, emb_table, W, b):
    idx = inputs.astype(jnp.int32)
    table_lin = emb_table.T.reshape(-1)
    offs = (jnp.arange(_EMBED_DIM, dtype=jnp.int32) * _VOCAB)[:, None] + idx[None, :]
    xta = _make_sc_gather()(table_lin, offs)
    wta = jnp.concatenate([W.T, b.reshape(1, _VOCAB)], axis=0)
    xt_spec = pl.BlockSpec((_KA, _BATCH), lambda j: (0, 0))
    lse_spec = pl.BlockSpec((1, _BATCH), lambda j: (0, 0))
    lse = pl.pallas_call(
        _lse_body,
        grid=(pl.cdiv(_VOCAB, _VTA),),
        in_specs=[pl.BlockSpec((_KA, _VTA), lambda j: (0, j)), xt_spec],
        out_specs=lse_spec,
        out_shape=jax.ShapeDtypeStruct((1, _BATCH), jnp.float32),
        scratch_shapes=[pltpu.VMEM((1, _BATCH), jnp.float32)],
    )(wta, xta)
    out_t = pl.pallas_call(
        _out_body,
        grid=(pl.cdiv(_VOCAB, _VT),),
        in_specs=[pl.BlockSpec((_KA, _VT), lambda j: (0, j)), xt_spec,
                  lse_spec],
        out_specs=pl.BlockSpec((_VT, _BATCH), lambda j: (j, 0)),
        out_shape=jax.ShapeDtypeStruct((_VOCAB, _BATCH), jnp.float32),
    )(wta, xta, lse)
    return out_t.T
